# baseline (speedup 1.0000x reference)
import jax
import jax.numpy as jnp
from jax.experimental import pallas as pl
from jax.experimental.pallas import tpu as pltpu

HIDDEN = 1024
FF = 4096
E = 8
TOPK = 2
BT = 2048


def _router_body(x_ref, gw_ref, logits_ref, w_ref, xb_ref):
    x = x_ref[...].astype(jnp.bfloat16)
    xb_ref[...] = x
    gw = gw_ref[...].astype(jnp.bfloat16)
    logits = jax.lax.dot_general(
        x, gw,
        dimension_numbers=(((1,), (1,)), ((), ())),
        preferred_element_type=jnp.float32,
    )
    logits_ref[...] = logits
    rw = jax.nn.softmax(logits, axis=-1)
    idx = jax.lax.broadcasted_iota(jnp.int32, rw.shape, 1)
    v1 = jnp.max(rw, axis=1, keepdims=True)
    i1 = jnp.min(jnp.where(rw == v1, idx, E), axis=1, keepdims=True)
    masked = jnp.where(idx == i1, -jnp.inf, rw)
    v2 = jnp.max(masked, axis=1, keepdims=True)
    i2 = jnp.min(jnp.where(masked == v2, idx, E), axis=1, keepdims=True)
    denom = v1 + v2
    w = jnp.where(idx == i1, v1, 0.0) + jnp.where(idx == i2, v2, 0.0)
    w_ref[...] = w / denom


def _router(x, gate_w):
    t = x.shape[0]
    return pl.pallas_call(
        _router_body,
        grid=(1,),
        in_specs=[
            pl.BlockSpec((t, HIDDEN), lambda i: (0, 0)),
            pl.BlockSpec((E, HIDDEN), lambda i: (0, 0)),
        ],
        out_specs=[
            pl.BlockSpec((t, E), lambda i: (0, 0)),
            pl.BlockSpec((t, E), lambda i: (0, 0)),
            pl.BlockSpec((t, HIDDEN), lambda i: (0, 0)),
        ],
        out_shape=[
            jax.ShapeDtypeStruct((t, E), jnp.float32),
            jax.ShapeDtypeStruct((t, E), jnp.float32),
            jax.ShapeDtypeStruct((t, HIDDEN), jnp.bfloat16),
        ],
    )(x, gate_w)


def _cast_body(w1_ref, w2_ref, o1_ref, o2_ref):
    o1_ref[...] = w1_ref[...].astype(jnp.bfloat16)
    o2_ref[...] = w2_ref[...].astype(jnp.bfloat16)


def _cast_weights(w1, w2):
    return pl.pallas_call(
        _cast_body,
        grid=(2 * E,),
        in_specs=[
            pl.BlockSpec((1, HIDDEN // 2, FF), lambda i: (i // 2, i % 2, 0)),
            pl.BlockSpec((1, FF // 2, HIDDEN), lambda i: (i // 2, i % 2, 0)),
        ],
        out_specs=[
            pl.BlockSpec((1, HIDDEN // 2, FF), lambda i: (i // 2, i % 2, 0)),
            pl.BlockSpec((1, FF // 2, HIDDEN), lambda i: (i // 2, i % 2, 0)),
        ],
        out_shape=[
            jax.ShapeDtypeStruct(w1.shape, jnp.bfloat16),
            jax.ShapeDtypeStruct(w2.shape, jnp.bfloat16),
        ],
        compiler_params=pltpu.CompilerParams(
            dimension_semantics=("arbitrary",),
        ),
    )(w1, w2)


def _ffn_body(x_ref, w1_ref, w2_ref, wts_ref, bias_ref, out_ref):
    e = pl.program_id(1)
    f = pl.program_id(2)

    @pl.when((e == 0) & (f == 0))
    def _():
        out_ref[...] = jnp.broadcast_to(bias_ref[...], out_ref.shape)

    h = jax.lax.dot_general(
        x_ref[...], w1_ref[0],
        dimension_numbers=(((1,), (0,)), ((), ())),
        preferred_element_type=jnp.float32,
    )
    hb = h.astype(jnp.bfloat16)
    gb = hb * (0.5 + 0.5 * jax.lax.erf(hb * jnp.bfloat16(0.70710678)))
    o = jax.lax.dot_general(
        gb, w2_ref[0],
        dimension_numbers=(((1,), (0,)), ((), ())),
        preferred_element_type=jnp.float32,
    )
    out_ref[...] += o * wts_ref[0]


def _ffn(xb, w1b, w2b, wts, bias2d):
    t = xb.shape[0]
    grid = (t // BT, E, 2)
    return pl.pallas_call(
        _ffn_body,
        grid=grid,
        in_specs=[
            pl.BlockSpec((BT, HIDDEN), lambda i, e, f: (i, 0)),
            pl.BlockSpec((1, HIDDEN, FF // 2), lambda i, e, f: (e, 0, f)),
            pl.BlockSpec((1, FF // 2, HIDDEN), lambda i, e, f: (e, f, 0)),
            pl.BlockSpec((1, BT, 1), lambda i, e, f: (e, i, 0)),
            pl.BlockSpec((1, HIDDEN), lambda i, e, f: (0, 0)),
        ],
        out_specs=pl.BlockSpec((BT, HIDDEN), lambda i, e, f: (i, 0)),
        out_shape=jax.ShapeDtypeStruct((t, HIDDEN), jnp.float32),
        compiler_params=pltpu.CompilerParams(
            dimension_semantics=("parallel", "arbitrary", "arbitrary"),
        ),
    )(xb, w1b, w2b, wts, bias2d)


def kernel(hidden_states, gate_w, w1, w2, bias):
    b, s, d = hidden_states.shape
    x = hidden_states.reshape(-1, d)
    t = x.shape[0]

    router_logits, wmat, xb = _router(x, gate_w)
    w1b, w2b = _cast_weights(w1, w2)

    wts = wmat.T.reshape(E, t, 1)
    bias2d = bias.reshape(1, HIDDEN)

    final = _ffn(xb, w1b, w2b, wts, bias2d)
    return (final.reshape(b, s, d), router_logits)


# speedup vs baseline: 1.0257x; 1.0257x over previous
import jax
import jax.numpy as jnp
from jax.experimental import pallas as pl
from jax.experimental.pallas import tpu as pltpu

HIDDEN = 1024
FF = 4096
E = 8
TOPK = 2
BT = 512


def _router_body(x_ref, gw_ref, logits_ref, w_ref, xb_ref):
    x = x_ref[...].astype(jnp.bfloat16)
    xb_ref[...] = x
    gw = gw_ref[...].astype(jnp.bfloat16)
    logits = jax.lax.dot_general(
        x, gw,
        dimension_numbers=(((1,), (1,)), ((), ())),
        preferred_element_type=jnp.float32,
    )
    logits_ref[...] = logits
    rw = jax.nn.softmax(logits, axis=-1)
    idx = jax.lax.broadcasted_iota(jnp.int32, rw.shape, 1)
    v1 = jnp.max(rw, axis=1, keepdims=True)
    i1 = jnp.min(jnp.where(rw == v1, idx, E), axis=1, keepdims=True)
    masked = jnp.where(idx == i1, -jnp.inf, rw)
    v2 = jnp.max(masked, axis=1, keepdims=True)
    i2 = jnp.min(jnp.where(masked == v2, idx, E), axis=1, keepdims=True)
    denom = v1 + v2
    w = jnp.where(idx == i1, v1, 0.0) + jnp.where(idx == i2, v2, 0.0)
    w_ref[...] = w / denom


def _router(x, gate_w):
    t = x.shape[0]
    return pl.pallas_call(
        _router_body,
        grid=(1,),
        in_specs=[
            pl.BlockSpec((t, HIDDEN), lambda i: (0, 0)),
            pl.BlockSpec((E, HIDDEN), lambda i: (0, 0)),
        ],
        out_specs=[
            pl.BlockSpec((t, E), lambda i: (0, 0)),
            pl.BlockSpec((t, E), lambda i: (0, 0)),
            pl.BlockSpec((t, HIDDEN), lambda i: (0, 0)),
        ],
        out_shape=[
            jax.ShapeDtypeStruct((t, E), jnp.float32),
            jax.ShapeDtypeStruct((t, E), jnp.float32),
            jax.ShapeDtypeStruct((t, HIDDEN), jnp.bfloat16),
        ],
    )(x, gate_w)


def _cast_body(w1_ref, w2_ref, o1_ref, o2_ref):
    o1_ref[...] = w1_ref[...].astype(jnp.bfloat16)
    o2_ref[...] = w2_ref[...].astype(jnp.bfloat16)


def _cast_weights(w1, w2):
    return pl.pallas_call(
        _cast_body,
        grid=(2 * E,),
        in_specs=[
            pl.BlockSpec((1, HIDDEN // 2, FF), lambda i: (i // 2, i % 2, 0)),
            pl.BlockSpec((1, FF // 2, HIDDEN), lambda i: (i // 2, i % 2, 0)),
        ],
        out_specs=[
            pl.BlockSpec((1, HIDDEN // 2, FF), lambda i: (i // 2, i % 2, 0)),
            pl.BlockSpec((1, FF // 2, HIDDEN), lambda i: (i // 2, i % 2, 0)),
        ],
        out_shape=[
            jax.ShapeDtypeStruct(w1.shape, jnp.bfloat16),
            jax.ShapeDtypeStruct(w2.shape, jnp.bfloat16),
        ],
        compiler_params=pltpu.CompilerParams(
            dimension_semantics=("arbitrary",),
        ),
    )(w1, w2)


def _ffn_body(x_ref, w1_ref, w2_ref, wts_ref, bias_ref, out_ref):
    e = pl.program_id(1)

    @pl.when(e == 0)
    def _():
        out_ref[...] = jnp.broadcast_to(bias_ref[...], out_ref.shape)

    h = jax.lax.dot_general(
        x_ref[...], w1_ref[0],
        dimension_numbers=(((1,), (0,)), ((), ())),
        preferred_element_type=jnp.float32,
    )
    h = 0.5 * h * (1.0 + jax.lax.erf(h * 0.7071067811865476))
    o = jax.lax.dot_general(
        h.astype(jnp.bfloat16), w2_ref[0],
        dimension_numbers=(((1,), (0,)), ((), ())),
        preferred_element_type=jnp.float32,
    )
    out_ref[...] += o * wts_ref[0]


def _ffn(xb, w1b, w2b, wts, bias2d):
    t = xb.shape[0]
    grid = (t // BT, E)
    return pl.pallas_call(
        _ffn_body,
        grid=grid,
        in_specs=[
            pl.BlockSpec((BT, HIDDEN), lambda i, e: (i, 0)),
            pl.BlockSpec((1, HIDDEN, FF), lambda i, e: (e, 0, 0)),
            pl.BlockSpec((1, FF, HIDDEN), lambda i, e: (e, 0, 0)),
            pl.BlockSpec((1, BT, 1), lambda i, e: (e, i, 0)),
            pl.BlockSpec((1, HIDDEN), lambda i, e: (0, 0)),
        ],
        out_specs=pl.BlockSpec((BT, HIDDEN), lambda i, e: (i, 0)),
        out_shape=jax.ShapeDtypeStruct((t, HIDDEN), jnp.float32),
        compiler_params=pltpu.CompilerParams(
            dimension_semantics=("parallel", "arbitrary"),
        ),
    )(xb, w1b, w2b, wts, bias2d)


def kernel(hidden_states, gate_w, w1, w2, bias):
    b, s, d = hidden_states.shape
    x = hidden_states.reshape(-1, d)
    t = x.shape[0]

    router_logits, wmat, xb = _router(x, gate_w)
    w1b, w2b = _cast_weights(w1, w2)

    wts = wmat.T.reshape(E, t, 1)
    bias2d = bias.reshape(1, HIDDEN)

    final = _ffn(xb, w1b, w2b, wts, bias2d)
    return (final.reshape(b, s, d), router_logits)
